# baseline (speedup 1.0000x reference)
import jax
import jax.numpy as jnp
from jax.experimental import pallas as pl
from jax.experimental.pallas import tpu as pltpu

_EPS = 1e-5
_CB = 16


def _cin_kernel(styles_ref, x_ref, g_ref, b_ref, o_ref):
    del styles_ref
    x = x_ref[...]
    n = x.shape[2] * x.shape[3]
    mean = jnp.sum(x, axis=(2, 3), keepdims=True) / n
    sq = jnp.sum(x * x, axis=(2, 3), keepdims=True) / n
    var = sq - mean * mean
    scale = jax.lax.rsqrt(var + _EPS) * g_ref[...]
    shift = b_ref[...] - mean * scale
    o_ref[...] = x * scale + shift


def kernel(x, styles, gamma, beta):
    B, C, H, W = x.shape
    S = gamma.shape[0]
    styles = styles.astype(jnp.int32)
    g4 = gamma.reshape(S, C, 1, 1)
    b4 = beta.reshape(S, C, 1, 1)

    grid_spec = pltpu.PrefetchScalarGridSpec(
        num_scalar_prefetch=1,
        grid=(B, C // _CB),
        in_specs=[
            pl.BlockSpec((1, _CB, H, W), lambda i, j, s: (i, j, 0, 0)),
            pl.BlockSpec((1, _CB, 1, 1), lambda i, j, s: (s[i], j, 0, 0)),
            pl.BlockSpec((1, _CB, 1, 1), lambda i, j, s: (s[i], j, 0, 0)),
        ],
        out_specs=pl.BlockSpec((1, _CB, H, W), lambda i, j, s: (i, j, 0, 0)),
    )
    return pl.pallas_call(
        _cin_kernel,
        out_shape=jax.ShapeDtypeStruct((B, C, H, W), x.dtype),
        grid_spec=grid_spec,
        compiler_params=pltpu.CompilerParams(
            dimension_semantics=("parallel", "parallel"),
        ),
        name="conditional_instance_norm",
    )(styles, x, g4, b4)


# speedup vs baseline: 1.0212x; 1.0212x over previous
import jax
import jax.numpy as jnp
from jax.experimental import pallas as pl
from jax.experimental.pallas import tpu as pltpu

_EPS = 1e-5
_CB = 32
_HB = _CB // 2


def _norm_half(x, g, b):
    n = x.shape[2] * x.shape[3]
    mean = jnp.sum(x, axis=(2, 3), keepdims=True) / n
    sq = jnp.sum(x * x, axis=(2, 3), keepdims=True) / n
    var = sq - mean * mean
    scale = jax.lax.rsqrt(var + _EPS) * g
    shift = b - mean * scale
    return x * scale + shift


def _cin_kernel(styles_ref, x0_ref, x1_ref, g_ref, b_ref, o_ref):
    del styles_ref
    o_ref[:, :_HB] = _norm_half(
        x0_ref[...], g_ref[:, :_HB], b_ref[:, :_HB]
    )
    o_ref[:, _HB:] = _norm_half(
        x1_ref[...], g_ref[:, _HB:], b_ref[:, _HB:]
    )


def kernel(x, styles, gamma, beta):
    B, C, H, W = x.shape
    S = gamma.shape[0]
    styles = styles.astype(jnp.int32)
    g4 = gamma.reshape(S, C, 1, 1)
    b4 = beta.reshape(S, C, 1, 1)

    grid_spec = pltpu.PrefetchScalarGridSpec(
        num_scalar_prefetch=1,
        grid=(B, C // _CB),
        in_specs=[
            pl.BlockSpec((1, _HB, H, W), lambda i, j, s: (i, 2 * j, 0, 0)),
            pl.BlockSpec((1, _HB, H, W), lambda i, j, s: (i, 2 * j + 1, 0, 0)),
            pl.BlockSpec((1, _CB, 1, 1), lambda i, j, s: (s[i], j, 0, 0)),
            pl.BlockSpec((1, _CB, 1, 1), lambda i, j, s: (s[i], j, 0, 0)),
        ],
        out_specs=pl.BlockSpec((1, _CB, H, W), lambda i, j, s: (i, j, 0, 0)),
    )
    return pl.pallas_call(
        _cin_kernel,
        out_shape=jax.ShapeDtypeStruct((B, C, H, W), x.dtype),
        grid_spec=grid_spec,
        compiler_params=pltpu.CompilerParams(
            dimension_semantics=("parallel", "parallel"),
        ),
        name="conditional_instance_norm",
    )(styles, x, x, g4, b4)


# speedup vs baseline: 1.0217x; 1.0005x over previous
import jax
import jax.numpy as jnp
from jax.experimental import pallas as pl
from jax.experimental.pallas import tpu as pltpu

_EPS = 1e-5
_CB = 32


def _cin_kernel(styles_ref, x_ref, g_ref, b_ref, o_ref):
    del styles_ref
    x = x_ref[...]
    n = x.shape[2] * x.shape[3]
    mean = jnp.sum(x, axis=(2, 3), keepdims=True) / n
    sq = jnp.sum(x * x, axis=(2, 3), keepdims=True) / n
    var = sq - mean * mean
    scale = jax.lax.rsqrt(var + _EPS) * g_ref[...]
    shift = b_ref[...] - mean * scale
    o_ref[...] = x * scale + shift


def kernel(x, styles, gamma, beta):
    B, C, H, W = x.shape
    S = gamma.shape[0]
    styles = styles.astype(jnp.int32)
    g4 = gamma.reshape(S, C, 1, 1)
    b4 = beta.reshape(S, C, 1, 1)

    grid_spec = pltpu.PrefetchScalarGridSpec(
        num_scalar_prefetch=1,
        grid=(B, C // _CB),
        in_specs=[
            pl.BlockSpec((1, _CB, H, W), lambda i, j, s: (i, j, 0, 0)),
            pl.BlockSpec((1, _CB, 1, 1), lambda i, j, s: (s[i], j, 0, 0)),
            pl.BlockSpec((1, _CB, 1, 1), lambda i, j, s: (s[i], j, 0, 0)),
        ],
        out_specs=pl.BlockSpec((1, _CB, H, W), lambda i, j, s: (i, j, 0, 0)),
    )
    return pl.pallas_call(
        _cin_kernel,
        out_shape=jax.ShapeDtypeStruct((B, C, H, W), x.dtype),
        grid_spec=grid_spec,
        compiler_params=pltpu.CompilerParams(
            dimension_semantics=("parallel", "parallel"),
        ),
        name="conditional_instance_norm",
    )(styles, x, g4, b4)
